# baseline (speedup 1.0000x reference)
import functools

import jax
import jax.numpy as jnp
from jax import lax
from jax.experimental import pallas as pl
from jax.experimental.pallas import tpu as pltpu
from jax.experimental.pallas import tpu_sc as plsc

_NC = 2
_NS = 16
_NW = _NC * _NS

_CH = 14
_L = 16


@functools.lru_cache(maxsize=None)
def _make_gather(V, D, B):
  assert D % _L == 0 and B % _NW == 0
  b_per_w = B // _NW
  n_tc = (V + 127) // 128
  c_max = n_tc - _CH
  W = _CH * 128
  i32max = jnp.iinfo(jnp.int32).max

  mesh = plsc.VectorSubcoreMesh(core_axis_name="c", subcore_axis_name="s")

  @functools.partial(
      pl.kernel,
      out_type=jax.ShapeDtypeStruct(((B + 1) * D,), jnp.float32),
      mesh=mesh,
      scratch_types=[
          pltpu.VMEM((b_per_w + _L,), jnp.int32),
          pltpu.VMEM((b_per_w + _L,), jnp.int32),
          pltpu.VMEM((D, W), jnp.float32),
          pltpu.VMEM((_L * D,), jnp.float32),
          pltpu.SemaphoreType.DMA,
          pltpu.SemaphoreType.DMA,
      ],
      compiler_params=pltpu.CompilerParams(
          use_tc_tiling_on_sc=True, needs_layout_passes=False
      ),
  )
  def k(tT_hbm, sidx_hbm, spos_hbm, out_hbm, idx_v, pos_v, buf, hbuf, ssem, wsem):
    wid = lax.axis_index("s") * _NC + lax.axis_index("c")
    base = wid * b_per_w
    pltpu.sync_copy(sidx_hbm.at[pl.ds(base, b_per_w)], idx_v.at[pl.ds(0, b_per_w)])
    pltpu.sync_copy(spos_hbm.at[pl.ds(base, b_per_w)], pos_v.at[pl.ds(0, b_per_w)])
    idx_v[pl.ds(b_per_w, _L)] = jnp.full((_L,), i32max, jnp.int32)
    iota = lax.iota(jnp.int32, _L)

    def step(s, carry):
      ptr, pc0 = carry
      active = ptr < b_per_w
      v = idx_v[pl.ds(ptr, _L)]
      pv = pos_v[pl.ds(ptr, _L)]
      c0 = jnp.minimum((v[0] // (128 * _CH)) * _CH, c_max)
      fetch = jnp.logical_and(active, c0 != pc0)

      @pl.when(fetch)
      def _():
        cp = pltpu.make_async_copy(
            tT_hbm.at[:, pl.ds(pl.multiple_of(c0 * 128, 128), W)],
            buf,
            ssem,
        )
        cp.start()
        cp.wait()

      off = c0 * 128
      hi = off + W
      n0 = plsc.all_reduce_population_count(v < hi)[0]

      @pl.when(active)
      def _():
        copies = []
        for kk in range(_L):
          col = jnp.full((_L,), jnp.clip(v[kk] - off, 0, W - 1), jnp.int32)
          for q in range(D // _L):
            vals = plsc.load_gather(buf, [iota + q * _L, col])
            hbuf[pl.ds(kk * D + q * _L, _L)] = vals
          dst_row = jnp.where(kk < n0, pv[kk], B)
          copies.append(
              pltpu.make_async_copy(
                  hbuf.at[pl.ds(kk * D, D)],
                  out_hbm.at[pl.ds(dst_row * D, D)],
                  wsem,
              )
          )
          copies[-1].start()
        for cp in copies:
          cp.wait()

      return (ptr + n0, c0)

    pl.loop(0, b_per_w, init_carry=(jnp.int32(0), jnp.int32(-1)))(step)

  return k


def kernel(t, index):
  V, D = t.shape
  (B,) = index.shape
  idx32 = index.astype(jnp.int32)
  sidx, spos = lax.sort_key_val(idx32, lax.iota(jnp.int32, B))
  flat = _make_gather(V, D, B)(t.T, sidx, spos)
  return flat[: B * D].reshape(B, D)


# speedup vs baseline: 1.1551x; 1.1551x over previous
import functools

import jax
import jax.numpy as jnp
from jax import lax
from jax.experimental import pallas as pl
from jax.experimental.pallas import tpu as pltpu
from jax.experimental.pallas import tpu_sc as plsc

_NC = 2
_NS = 16
_NW = _NC * _NS

_CH = 14
_L = 16


@functools.lru_cache(maxsize=None)
def _make_gather(V, D, B):
  assert D % _L == 0 and B % _NW == 0
  b_per_w = B // _NW
  n_tc = (V + 127) // 128
  c_max = n_tc - _CH
  W = _CH * 128
  i32max = jnp.iinfo(jnp.int32).max

  mesh = plsc.VectorSubcoreMesh(core_axis_name="c", subcore_axis_name="s")

  @functools.partial(
      pl.kernel,
      out_type=jax.ShapeDtypeStruct(((B + 1) * D,), jnp.float32),
      mesh=mesh,
      scratch_types=[
          pltpu.VMEM((b_per_w + _L,), jnp.int32),
          pltpu.VMEM((b_per_w + _L,), jnp.int32),
          pltpu.VMEM((D, W), jnp.float32),
          pltpu.VMEM((_L * D,), jnp.float32),
          pltpu.SemaphoreType.DMA,
          pltpu.SemaphoreType.DMA,
          pltpu.SemaphoreType.DMA,
      ],
      compiler_params=pltpu.CompilerParams(
          use_tc_tiling_on_sc=True, needs_layout_passes=False
      ),
  )
  def k(tT_hbm, sidx_hbm, spos_hbm, out_hbm, idx_v, pos_v, buf, hbuf,
        ssem, bsem, wsem):
    wid = lax.axis_index("s") * _NC + lax.axis_index("c")
    base = wid * b_per_w
    pltpu.sync_copy(sidx_hbm.at[pl.ds(base, b_per_w)], idx_v.at[pl.ds(0, b_per_w)])
    pltpu.sync_copy(spos_hbm.at[pl.ds(base, b_per_w)], pos_v.at[pl.ds(0, b_per_w)])
    idx_v[pl.ds(b_per_w, _L)] = jnp.full((_L,), i32max, jnp.int32)
    iota = lax.iota(jnp.int32, _L)

    def step(s, carry):
      ptr, pc0 = carry
      active = ptr < b_per_w
      v = idx_v[pl.ds(ptr, _L)]
      pv = pos_v[pl.ds(ptr, _L)]
      reuse = jnp.logical_and(pc0 >= 0, v[_L - 1] < (pc0 + _CH) * 128)
      c0 = jnp.where(reuse, pc0, jnp.minimum(v[0] // 128, c_max))
      fetch = jnp.logical_and(active, jnp.logical_not(reuse))
      off = c0 * 128
      off_a = pl.multiple_of(off, 128)

      def half_copy(r0, sem):
        return pltpu.make_async_copy(
            tT_hbm.at[pl.ds(r0, D // 2), pl.ds(off_a, W)],
            buf.at[pl.ds(r0, D // 2), :],
            sem,
        )

      @pl.when(fetch)
      def _():
        top = half_copy(0, ssem)
        bot = half_copy(D // 2, bsem)
        top.start()
        bot.start()
        top.wait()

      hi = off + W
      n0 = plsc.all_reduce_population_count(v < hi)[0]
      cols = [
          jnp.full((_L,), jnp.clip(v[kk] - off, 0, W - 1), jnp.int32)
          for kk in range(_L)
      ]

      @pl.when(active)
      def _():
        for kk in range(_L):
          for q in range(D // (2 * _L)):
            vals = plsc.load_gather(buf, [iota + q * _L, cols[kk]])
            hbuf[pl.ds(kk * D + q * _L, _L)] = vals

      @pl.when(fetch)
      def _():
        half_copy(D // 2, bsem).wait()

      @pl.when(active)
      def _():
        copies = []
        for kk in range(_L):
          for q in range(D // (2 * _L), D // _L):
            vals = plsc.load_gather(buf, [iota + q * _L, cols[kk]])
            hbuf[pl.ds(kk * D + q * _L, _L)] = vals
          dst_row = jnp.where(kk < n0, pv[kk], B)
          copies.append(
              pltpu.make_async_copy(
                  hbuf.at[pl.ds(kk * D, D)],
                  out_hbm.at[pl.ds(dst_row * D, D)],
                  wsem,
              )
          )
          copies[-1].start()
        for cp in copies:
          cp.wait()

      return (ptr + n0, c0)

    pl.loop(0, b_per_w, init_carry=(jnp.int32(0), jnp.int32(-1)))(step)

  return k


def kernel(t, index):
  V, D = t.shape
  (B,) = index.shape
  idx32 = index.astype(jnp.int32)
  sidx, spos = lax.sort_key_val(idx32, lax.iota(jnp.int32, B))
  flat = _make_gather(V, D, B)(t.T, sidx, spos)
  return flat[: B * D].reshape(B, D)
